# Initial kernel scaffold; baseline (speedup 1.0000x reference)
import jax
import jax.numpy as jnp
from jax.experimental import pallas as pl


def kernel(input_ids, token_embed):
    raise NotImplementedError("write your pallas kernel here")


# speedup vs baseline: 1.4239x; 1.4239x over previous
import functools

import jax
import jax.numpy as jnp
from jax import lax
from jax.experimental import pallas as pl
from jax.experimental.pallas import tpu as pltpu
from jax.experimental.pallas import tpu_sc as plsc

_NC = 2
_NS = 16
_NW = _NC * _NS


@functools.lru_cache(maxsize=None)
def _make_gather(V: int, D: int, B: int):
    b_per_w = B // _NW
    chunk = 64
    n_chunks = b_per_w // chunk
    mesh = plsc.VectorSubcoreMesh(core_axis_name="c", subcore_axis_name="s")

    @functools.partial(
        pl.kernel,
        mesh=mesh,
        out_type=jax.ShapeDtypeStruct((B, D), jnp.float32),
        scratch_types=[
            pltpu.VMEM((b_per_w,), jnp.int32),
            pltpu.VMEM((chunk, D), jnp.float32),
            pltpu.SemaphoreType.DMA,
        ],
    )
    def gather_kernel(ids_hbm, table_hbm, out_hbm, idx_v, rows_v, sem):
        wid = lax.axis_index("s") * _NC + lax.axis_index("c")
        base = wid * b_per_w
        pltpu.sync_copy(ids_hbm.at[pl.ds(base, b_per_w)], idx_v)
        for ch in range(n_chunks):
            pltpu.async_copy(
                table_hbm.at[idx_v.at[pl.ds(ch * chunk, chunk)]],
                rows_v,
                sem,
            ).wait()
            pltpu.sync_copy(
                rows_v, out_hbm.at[pl.ds(base + ch * chunk, chunk)]
            )

    return gather_kernel


def kernel(input_ids, token_embed):
    batch, seq = input_ids.shape
    vocab, d_model = token_embed.shape
    ids = input_ids.reshape(-1).astype(jnp.int32)
    out = _make_gather(vocab, d_model, batch * seq)(ids, token_embed)
    return out.reshape(batch, seq, d_model)


# speedup vs baseline: 1.4787x; 1.0385x over previous
import functools

import jax
import jax.numpy as jnp
from jax import lax
from jax.experimental import pallas as pl
from jax.experimental.pallas import tpu as pltpu
from jax.experimental.pallas import tpu_sc as plsc

_NC = 2
_NS = 16
_NW = _NC * _NS


@functools.lru_cache(maxsize=None)
def _make_gather(V: int, D: int, B: int):
    b_per_w = B // _NW
    chunk = 64
    n_chunks = b_per_w // chunk
    mesh = plsc.VectorSubcoreMesh(core_axis_name="c", subcore_axis_name="s")

    @functools.partial(
        pl.kernel,
        mesh=mesh,
        out_type=jax.ShapeDtypeStruct((B, D), jnp.float32),
        scratch_types=[
            pltpu.VMEM((b_per_w,), jnp.int32),
            pltpu.VMEM((chunk, D), jnp.float32),
            pltpu.VMEM((chunk, D), jnp.float32),
            pltpu.SemaphoreType.DMA,
            pltpu.SemaphoreType.DMA,
            pltpu.SemaphoreType.DMA,
            pltpu.SemaphoreType.DMA,
        ],
    )
    def gather_kernel(ids_hbm, table_hbm, out_hbm, idx_v, rows0, rows1,
                      gs0, gs1, ws0, ws1):
        wid = lax.axis_index("s") * _NC + lax.axis_index("c")
        base = wid * b_per_w
        pltpu.sync_copy(ids_hbm.at[pl.ds(base, b_per_w)], idx_v)
        bufs, gsems, wsems = (rows0, rows1), (gs0, gs1), (ws0, ws1)
        gcp = [None, None]
        wcp = [None, None]
        gcp[0] = pltpu.async_copy(
            table_hbm.at[idx_v.at[pl.ds(0, chunk)]], bufs[0], gsems[0])
        for ch in range(n_chunks):
            cur = ch % 2
            nxt = 1 - cur
            if ch + 1 < n_chunks:
                if wcp[nxt] is not None:
                    wcp[nxt].wait()
                gcp[nxt] = pltpu.async_copy(
                    table_hbm.at[idx_v.at[pl.ds((ch + 1) * chunk, chunk)]],
                    bufs[nxt], gsems[nxt])
            gcp[cur].wait()
            wcp[cur] = pltpu.async_copy(
                bufs[cur], out_hbm.at[pl.ds(base + ch * chunk, chunk)],
                wsems[cur])
        wcp[(n_chunks - 2) % 2].wait()
        wcp[(n_chunks - 1) % 2].wait()

    return gather_kernel


def kernel(input_ids, token_embed):
    batch, seq = input_ids.shape
    vocab, d_model = token_embed.shape
    ids = input_ids.reshape(-1).astype(jnp.int32)
    out = _make_gather(vocab, d_model, batch * seq)(ids, token_embed)
    return out.reshape(batch, seq, d_model)
